# Initial kernel scaffold; baseline (speedup 1.0000x reference)
import jax
import jax.numpy as jnp
from jax.experimental import pallas as pl


def kernel(hidden_states, W1, b1, W2, b2):
    raise NotImplementedError("write your pallas kernel here")


# speedup vs baseline: 1.6000x; 1.6000x over previous
import functools

import jax
import jax.numpy as jnp
from jax.experimental import pallas as pl
from jax.experimental.pallas import tpu as pltpu

D_MODEL = 768
D_HID = 384
N_EXP = 64
TOK_BLOCK = 1024


def _router_block(x_ref, w1_ref, b1_ref, w2_ref, b2_ref,
                  gates_ref, idx_ref, probs_ref):
    x = x_ref[...]
    h = jnp.dot(x, w1_ref[...], preferred_element_type=jnp.float32,
                precision=jax.lax.Precision.DEFAULT)
    h = jnp.maximum(h + b1_ref[...], 0.0)
    logits = jnp.dot(h, w2_ref[...], preferred_element_type=jnp.float32,
                     precision=jax.lax.Precision.DEFAULT)
    logits = logits + b2_ref[...]
    m = jnp.max(logits, axis=-1, keepdims=True)
    e = jnp.exp(logits - m)
    s = jnp.sum(e, axis=-1, keepdims=True)
    p = e / s
    probs_ref[...] = p

    iota = jax.lax.broadcasted_iota(jnp.int32, p.shape, 1)
    v1 = jnp.max(p, axis=-1, keepdims=True)
    i1 = jnp.min(jnp.where(p >= v1, iota, N_EXP), axis=-1, keepdims=True)
    pm = jnp.where(iota == i1, -1.0, p)
    v2 = jnp.max(pm, axis=-1, keepdims=True)
    i2 = jnp.min(jnp.where(pm >= v2, iota, N_EXP), axis=-1, keepdims=True)
    denom = v1 + v2 + 1e-8
    gates_ref[...] = jnp.concatenate([v1 / denom, v2 / denom], axis=-1)
    idx_ref[...] = jnp.concatenate([i1, i2], axis=-1)


@functools.partial(jax.jit, static_argnames=())
def kernel(hidden_states, W1, b1, W2, b2):
    n_tok = hidden_states.shape[0]
    grid = (n_tok // TOK_BLOCK,)
    b1r = b1.reshape(1, D_HID)
    b2r = b2.reshape(1, N_EXP)
    out_shapes = (
        jax.ShapeDtypeStruct((n_tok, 2), jnp.float32),
        jax.ShapeDtypeStruct((n_tok, 2), jnp.int32),
        jax.ShapeDtypeStruct((n_tok, N_EXP), jnp.float32),
    )
    full = lambda shape: pl.BlockSpec(shape, lambda i: (0, 0))
    gates, idx, probs = pl.pallas_call(
        _router_block,
        grid=grid,
        in_specs=[
            pl.BlockSpec((TOK_BLOCK, D_MODEL), lambda i: (i, 0)),
            full((D_MODEL, D_HID)),
            full((1, D_HID)),
            full((D_HID, N_EXP)),
            full((1, N_EXP)),
        ],
        out_specs=(
            pl.BlockSpec((TOK_BLOCK, 2), lambda i: (i, 0)),
            pl.BlockSpec((TOK_BLOCK, 2), lambda i: (i, 0)),
            pl.BlockSpec((TOK_BLOCK, N_EXP), lambda i: (i, 0)),
        ),
        out_shape=out_shapes,
        compiler_params=pltpu.CompilerParams(
            dimension_semantics=("parallel",),
        ),
    )(hidden_states, W1, b1r, W2, b2r)
    return (gates, idx, probs)


# speedup vs baseline: 1.7152x; 1.0720x over previous
import functools

import jax
import jax.numpy as jnp
from jax.experimental import pallas as pl
from jax.experimental.pallas import tpu as pltpu

D_MODEL = 768
D_HID = 384
N_EXP = 64
TOK_BLOCK = 2048


def _router_block(x_ref, w1_ref, b1_ref, w2_ref, b2_ref,
                  gates_ref, idx_ref, probs_ref):
    x = x_ref[...]
    h = jnp.dot(x, w1_ref[...], preferred_element_type=jnp.float32,
                precision=jax.lax.Precision.DEFAULT)
    h = jnp.maximum(h + b1_ref[...], 0.0)
    logits = jnp.dot(h, w2_ref[...], preferred_element_type=jnp.float32,
                     precision=jax.lax.Precision.DEFAULT)
    logits = logits + b2_ref[...]
    m = jnp.max(logits, axis=-1, keepdims=True)
    e = jnp.exp(logits - m)
    s = jnp.sum(e, axis=-1, keepdims=True)
    p = e / s
    probs_ref[...] = p

    iota = jax.lax.broadcasted_iota(jnp.int32, p.shape, 1)
    v1 = jnp.max(p, axis=-1, keepdims=True)
    i1 = jnp.min(jnp.where(p >= v1, iota, N_EXP), axis=-1, keepdims=True)
    pm = jnp.where(iota == i1, -1.0, p)
    v2 = jnp.max(pm, axis=-1, keepdims=True)
    i2 = jnp.min(jnp.where(pm >= v2, iota, N_EXP), axis=-1, keepdims=True)
    denom = v1 + v2 + 1e-8
    gates_ref[...] = jnp.concatenate([v1 / denom, v2 / denom], axis=-1)
    idx_ref[...] = jnp.concatenate([i1, i2], axis=-1)


@functools.partial(jax.jit, static_argnames=())
def kernel(hidden_states, W1, b1, W2, b2):
    n_tok = hidden_states.shape[0]
    grid = (n_tok // TOK_BLOCK,)
    b1r = b1.reshape(1, D_HID)
    b2r = b2.reshape(1, N_EXP)
    out_shapes = (
        jax.ShapeDtypeStruct((n_tok, 2), jnp.float32),
        jax.ShapeDtypeStruct((n_tok, 2), jnp.int32),
        jax.ShapeDtypeStruct((n_tok, N_EXP), jnp.float32),
    )
    full = lambda shape: pl.BlockSpec(shape, lambda i: (0, 0))
    gates, idx, probs = pl.pallas_call(
        _router_block,
        grid=grid,
        in_specs=[
            pl.BlockSpec((TOK_BLOCK, D_MODEL), lambda i: (i, 0)),
            full((D_MODEL, D_HID)),
            full((1, D_HID)),
            full((D_HID, N_EXP)),
            full((1, N_EXP)),
        ],
        out_specs=(
            pl.BlockSpec((TOK_BLOCK, 2), lambda i: (i, 0)),
            pl.BlockSpec((TOK_BLOCK, 2), lambda i: (i, 0)),
            pl.BlockSpec((TOK_BLOCK, N_EXP), lambda i: (i, 0)),
        ),
        out_shape=out_shapes,
        compiler_params=pltpu.CompilerParams(
            dimension_semantics=("parallel",),
        ),
    )(hidden_states, W1, b1r, W2, b2r)
    return (gates, idx, probs)
